# baseline (speedup 1.0000x reference)
import jax
import jax.numpy as jnp
from jax.experimental import pallas as pl
from jax.experimental.pallas import tpu as pltpu

B, D, T = 8, 128, 4096
NQ, CS = 30, 1024
N = B * T
TILE = 2048
NT = N // TILE


def _tree_sum_lanes(a):
    w = a.shape[-1]
    while w > 1:
        w //= 2
        a = a[..., :w] + a[..., w:2 * w]
    return a


def _scnorm_kernel(cb_ref, sc_ref):
    cb = cb_ref[0]
    cc = cb * cb
    acc = cc[:, 0:8]
    for v in range(1, 16):
        acc = acc + cc[:, 8 * v:8 * v + 8]
    acc = acc[:, 0:4] + acc[:, 4:8]
    acc = acc[:, 0:2] + acc[:, 2:4]
    sc_ref[...] = (acc[:, 0:1] + acc[:, 1:2]).reshape(1, 1, CS)


def _rvq_kernel(x_ref, cb_ref, sc_ref, out_ref, loss_ref, res_ref):
    j = pl.program_id(1)

    @pl.when(j == 0)
    def _init():
        res_ref[...] = x_ref[...]
        loss_ref[...] = jnp.zeros_like(loss_ref)

    res = res_ref[...]
    cb = cb_ref[0]
    sc = sc_ref[0]

    hi = cb.astype(jnp.bfloat16)
    mm2 = jax.lax.dot_general(res.astype(jnp.bfloat16) * jnp.bfloat16(-2.0),
                              hi, (((1,), (1,)), ((), ())),
                              preferred_element_type=jnp.float32)
    sx = jnp.sum(res * res, axis=1, keepdims=True)
    d2 = (sx + mm2) + sc

    iota = jax.lax.broadcasted_iota(jnp.int32, (TILE, CS), 1)
    dmin = jnp.min(d2, axis=1, keepdims=True)
    idx = jnp.min(jnp.where(d2 == dmin, iota, CS), axis=1)
    oh = (iota == idx[:, None]).astype(jnp.bfloat16)
    r1 = cb - hi.astype(jnp.float32)
    lo1 = r1.astype(jnp.bfloat16)
    lo2 = (r1 - lo1.astype(jnp.float32)).astype(jnp.bfloat16)
    dn = (((1,), (0,)), ((), ()))
    q_hi = jax.lax.dot_general(oh, hi, dn, preferred_element_type=jnp.float32)
    q_l1 = jax.lax.dot_general(oh, lo1, dn, preferred_element_type=jnp.float32)
    q_l2 = jax.lax.dot_general(oh, lo2, dn, preferred_element_type=jnp.float32)
    q = (q_hi + q_l1) + q_l2

    loss_ref[...] += jnp.sum((q - res) * (q - res), axis=0)[None, None, :]
    res_ref[...] = res - q

    @pl.when(j == NQ - 1)
    def _fin():
        out_ref[...] = x_ref[...] - res_ref[...]


@jax.jit
def kernel(x, codebooks):
    xr = jnp.transpose(x, (0, 2, 1)).reshape(N, D)
    sc_all = pl.pallas_call(
        _scnorm_kernel,
        grid=(NQ,),
        in_specs=[pl.BlockSpec((1, CS, D), lambda j: (j, 0, 0))],
        out_specs=pl.BlockSpec((1, 1, CS), lambda j: (j, 0, 0)),
        out_shape=jax.ShapeDtypeStruct((NQ, 1, CS), jnp.float32),
    )(codebooks)
    out, loss = pl.pallas_call(
        _rvq_kernel,
        grid=(NT, NQ),
        in_specs=[
            pl.BlockSpec((TILE, D), lambda i, j: (i, 0)),
            pl.BlockSpec((1, CS, D), lambda i, j: (j, 0, 0)),
            pl.BlockSpec((1, 1, CS), lambda i, j: (j, 0, 0)),
        ],
        out_specs=[
            pl.BlockSpec((TILE, D), lambda i, j: (i, 0)),
            pl.BlockSpec((1, 1, D), lambda i, j: (i, 0, 0)),
        ],
        out_shape=[
            jax.ShapeDtypeStruct((N, D), jnp.float32),
            jax.ShapeDtypeStruct((NT, 1, D), jnp.float32),
        ],
        scratch_shapes=[pltpu.VMEM((TILE, D), jnp.float32)],
    )(xr, codebooks, sc_all)
    quantized = jnp.transpose(out.reshape(B, T, D), (0, 2, 1))
    commit_loss = jnp.sum(loss) / jnp.float32(N * D)
    return quantized, commit_loss


# speedup vs baseline: 2.5501x; 2.5501x over previous
import jax
import jax.numpy as jnp
from jax.experimental import pallas as pl
from jax.experimental.pallas import tpu as pltpu

B, D, T = 8, 128, 4096
NQ, CS = 30, 1024
N = B * T
TILE = 2048
NT = N // TILE


def _tree_sum_lanes(a):
    w = a.shape[-1]
    while w > 1:
        w //= 2
        a = a[..., :w] + a[..., w:2 * w]
    return a


def _scnorm_kernel(cb_ref, sc_ref):
    cb = cb_ref[0]
    cc = cb * cb
    acc = cc[:, 0:8]
    for v in range(1, 16):
        acc = acc + cc[:, 8 * v:8 * v + 8]
    acc = acc[:, 0:4] + acc[:, 4:8]
    acc = acc[:, 0:2] + acc[:, 2:4]
    sc_ref[...] = (acc[:, 0:1] + acc[:, 1:2]).reshape(1, 1, CS)


def _rvq_kernel(x_ref, cb_ref, sc_ref, out_ref, loss_ref, res_ref):
    j = pl.program_id(1)

    @pl.when(j == 0)
    def _init():
        res_ref[...] = x_ref[...]
        loss_ref[...] = jnp.zeros_like(loss_ref)

    res = res_ref[...]
    cb = cb_ref[0]
    sc = sc_ref[0]

    hi = cb.astype(jnp.bfloat16)
    mm2 = jax.lax.dot_general(res.astype(jnp.bfloat16) * jnp.bfloat16(-2.0),
                              hi, (((1,), (1,)), ((), ())),
                              preferred_element_type=jnp.float32)
    rt = res.T
    rrt = rt * rt
    acc = rrt[0:8, :]
    for v in range(1, 16):
        acc = acc + rrt[8 * v:8 * v + 8, :]
    acc = acc[0:4, :] + acc[4:8, :]
    acc = acc[0:2, :] + acc[2:4, :]
    sx = (acc[0:1, :] + acc[1:2, :]).T
    d2 = (sx + mm2) + sc

    iota = jax.lax.broadcasted_iota(jnp.int32, (TILE, CS), 1)
    dmin = jnp.min(d2, axis=1, keepdims=True)
    idx = jnp.min(jnp.where(d2 == dmin, iota, CS), axis=1)
    oh = (iota == idx[:, None]).astype(jnp.bfloat16)
    r1 = cb - hi.astype(jnp.float32)
    lo1 = r1.astype(jnp.bfloat16)
    lo2 = (r1 - lo1.astype(jnp.float32)).astype(jnp.bfloat16)
    dn = (((1,), (0,)), ((), ()))
    q_hi = jax.lax.dot_general(oh, hi, dn, preferred_element_type=jnp.float32)
    q_l1 = jax.lax.dot_general(oh, lo1, dn, preferred_element_type=jnp.float32)
    q_l2 = jax.lax.dot_general(oh, lo2, dn, preferred_element_type=jnp.float32)
    q = (q_hi + q_l1) + q_l2

    loss_ref[...] += jnp.sum((q - res) * (q - res), axis=0)[None, None, :]
    res_ref[...] = res - q

    @pl.when(j == NQ - 1)
    def _fin():
        out_ref[...] = x_ref[...] - res_ref[...]


@jax.jit
def kernel(x, codebooks):
    xr = jnp.transpose(x, (0, 2, 1)).reshape(N, D)
    sc_all = pl.pallas_call(
        _scnorm_kernel,
        grid=(NQ,),
        in_specs=[pl.BlockSpec((1, CS, D), lambda j: (j, 0, 0))],
        out_specs=pl.BlockSpec((1, 1, CS), lambda j: (j, 0, 0)),
        out_shape=jax.ShapeDtypeStruct((NQ, 1, CS), jnp.float32),
    )(codebooks)
    out, loss = pl.pallas_call(
        _rvq_kernel,
        grid=(NT, NQ),
        in_specs=[
            pl.BlockSpec((TILE, D), lambda i, j: (i, 0)),
            pl.BlockSpec((1, CS, D), lambda i, j: (j, 0, 0)),
            pl.BlockSpec((1, 1, CS), lambda i, j: (j, 0, 0)),
        ],
        out_specs=[
            pl.BlockSpec((TILE, D), lambda i, j: (i, 0)),
            pl.BlockSpec((1, 1, D), lambda i, j: (i, 0, 0)),
        ],
        out_shape=[
            jax.ShapeDtypeStruct((N, D), jnp.float32),
            jax.ShapeDtypeStruct((NT, 1, D), jnp.float32),
        ],
        scratch_shapes=[pltpu.VMEM((TILE, D), jnp.float32)],
    )(xr, codebooks, sc_all)
    quantized = jnp.transpose(out.reshape(B, T, D), (0, 2, 1))
    commit_loss = jnp.sum(loss) / jnp.float32(N * D)
    return quantized, commit_loss


# speedup vs baseline: 3.0242x; 1.1859x over previous
import jax
import jax.numpy as jnp
from jax.experimental import pallas as pl
from jax.experimental.pallas import tpu as pltpu

B, D, T = 8, 128, 4096
NQ, CS = 30, 1024
N = B * T
TILE = 2048
NT = N // TILE


def _tree_sum_lanes(a):
    w = a.shape[-1]
    while w > 1:
        w //= 2
        a = a[..., :w] + a[..., w:2 * w]
    return a


def _scnorm_kernel(cb_ref, sc_ref):
    cb = cb_ref[0]
    cc = cb * cb
    acc = cc[:, 0:8]
    for v in range(1, 16):
        acc = acc + cc[:, 8 * v:8 * v + 8]
    acc = acc[:, 0:4] + acc[:, 4:8]
    acc = acc[:, 0:2] + acc[:, 2:4]
    sc_ref[...] = (acc[:, 0:1] + acc[:, 1:2]).reshape(1, 1, CS)


def _rvq_kernel(x_ref, cb_ref, sc_ref, out_ref, loss_ref, res_ref):
    j = pl.program_id(1)

    @pl.when(j == 0)
    def _init():
        res_ref[...] = x_ref[...].T
        loss_ref[...] = jnp.zeros_like(loss_ref)

    rest = res_ref[...]
    cb = cb_ref[0]
    sc = sc_ref[0]

    hi = cb.astype(jnp.bfloat16)
    mm2 = jax.lax.dot_general(rest.astype(jnp.bfloat16) * jnp.bfloat16(-2.0),
                              hi, (((0,), (1,)), ((), ())),
                              preferred_element_type=jnp.float32)
    rrt = rest * rest
    acc = rrt[0:8, :]
    for v in range(1, 16):
        acc = acc + rrt[8 * v:8 * v + 8, :]
    acc = acc[0:4, :] + acc[4:8, :]
    acc = acc[0:2, :] + acc[2:4, :]
    sx = (acc[0:1, :] + acc[1:2, :]).T
    d2 = (sx + mm2) + sc

    iota = jax.lax.broadcasted_iota(jnp.int32, (TILE, CS), 1)
    dmin = jnp.min(d2, axis=1, keepdims=True)
    idx = jnp.min(jnp.where(d2 == dmin, iota, CS), axis=1)
    oh = (iota == idx[:, None]).astype(jnp.bfloat16)
    r1 = cb - hi.astype(jnp.float32)
    lo1 = r1.astype(jnp.bfloat16)
    lo2 = (r1 - lo1.astype(jnp.float32)).astype(jnp.bfloat16)
    dn = (((0,), (1,)), ((), ()))
    qt_hi = jax.lax.dot_general(hi, oh, dn, preferred_element_type=jnp.float32)
    qt_l1 = jax.lax.dot_general(lo1, oh, dn, preferred_element_type=jnp.float32)
    qt_l2 = jax.lax.dot_general(lo2, oh, dn, preferred_element_type=jnp.float32)
    qt = (qt_hi + qt_l1) + qt_l2

    loss_ref[...] += jnp.sum((qt - rest) * (qt - rest), axis=1)[None, None, :]
    res_ref[...] = rest - qt

    @pl.when(j == NQ - 1)
    def _fin():
        out_ref[...] = x_ref[...] - res_ref[...].T


@jax.jit
def kernel(x, codebooks):
    xr = jnp.transpose(x, (0, 2, 1)).reshape(N, D)
    sc_all = pl.pallas_call(
        _scnorm_kernel,
        grid=(NQ,),
        in_specs=[pl.BlockSpec((1, CS, D), lambda j: (j, 0, 0))],
        out_specs=pl.BlockSpec((1, 1, CS), lambda j: (j, 0, 0)),
        out_shape=jax.ShapeDtypeStruct((NQ, 1, CS), jnp.float32),
    )(codebooks)
    out, loss = pl.pallas_call(
        _rvq_kernel,
        grid=(NT, NQ),
        in_specs=[
            pl.BlockSpec((TILE, D), lambda i, j: (i, 0)),
            pl.BlockSpec((1, CS, D), lambda i, j: (j, 0, 0)),
            pl.BlockSpec((1, 1, CS), lambda i, j: (j, 0, 0)),
        ],
        out_specs=[
            pl.BlockSpec((TILE, D), lambda i, j: (i, 0)),
            pl.BlockSpec((1, 1, D), lambda i, j: (i, 0, 0)),
        ],
        out_shape=[
            jax.ShapeDtypeStruct((N, D), jnp.float32),
            jax.ShapeDtypeStruct((NT, 1, D), jnp.float32),
        ],
        scratch_shapes=[pltpu.VMEM((D, TILE), jnp.float32)],
    )(xr, codebooks, sc_all)
    quantized = jnp.transpose(out.reshape(B, T, D), (0, 2, 1))
    commit_loss = jnp.sum(loss) / jnp.float32(N * D)
    return quantized, commit_loss
